# baseline (speedup 1.0000x reference)
import functools

import jax
import jax.numpy as jnp
from jax import lax
from jax.experimental import pallas as pl
from jax.experimental.pallas import tpu as pltpu
from jax.experimental.pallas import tpu_sc as plsc

_PREFIX = 32768
_S = 64
_CACHE_COLS = _PREFIX + _S * 16
_OUT_COLS = _PREFIX + _S
_NEG = jnp.finfo(jnp.float32).min
_SC_ROWS = 16
_BLK = 8
_A_STEPS = (_S - _SC_ROWS) // _BLK
_B_STEPS = _SC_ROWS // _BLK


@functools.partial(
    pl.kernel,
    out_type=jax.ShapeDtypeStruct((_SC_ROWS, _CACHE_COLS), jnp.bool_),
    mesh=plsc.VectorSubcoreMesh(core_axis_name="c", subcore_axis_name="s"),
    scratch_types=[
        pltpu.VMEM((1,), jnp.int32),
        pltpu.VMEM((1, _CACHE_COLS), jnp.bool_),
        pltpu.SemaphoreType.DMA,
    ],
)
def _sc_gather(table_hbm, idx_hbm, out_hbm, idx_v, row_v, sem):
    wid = lax.axis_index("s") * 2 + lax.axis_index("c")

    @pl.when(wid < _SC_ROWS)
    def _():
        pltpu.sync_copy(idx_hbm.at[wid], idx_v)
        pltpu.async_copy(table_hbm.at[idx_v], row_v, sem).wait()
        pltpu.sync_copy(row_v, out_hbm.at[pl.ds(wid, 1)])


def _eye_mask(row0, nrows):
    ri = lax.broadcasted_iota(jnp.int32, (nrows, _S), 0) + row0
    ci = lax.broadcasted_iota(jnp.int32, (nrows, _S), 1)
    return ri == ci


def _a_body(parents, r0, r1, r2, r3, r4, r5, r6, r7, out_ref):
    del parents
    zero = jnp.float32(0.0)
    neg = jnp.float32(_NEG)
    i = pl.program_id(0)
    ci = lax.iota(jnp.int32, _S)
    for k, r in enumerate((r0, r1, r2, r3, r4, r5, r6, r7)):
        out_ref[k, :_PREFIX] = jnp.where(r[0, 0, :_PREFIX], zero, neg)
        out_ref[k, _PREFIX:] = jnp.where(ci == _SC_ROWS + _BLK * i + k, zero, neg)


def _b_body(g_ref, alias_ref, out_ref):
    del alias_ref
    zero = jnp.float32(0.0)
    neg = jnp.float32(_NEG)
    i = pl.program_id(0)
    mask = jnp.concatenate(
        [g_ref[:, :_PREFIX], _eye_mask(_BLK * i, _BLK)], axis=1
    )
    out_ref[...] = jnp.where(mask, zero, neg)


def kernel(parent_indices, tree_mask_cache, eye_block):
    del eye_block
    cache = tree_mask_cache.reshape(_S, _CACHE_COLS)
    cache3 = tree_mask_cache.reshape(_S, 1, _CACHE_COLS)
    parents = parent_indices.reshape(_S)
    idx_sc = parent_indices.reshape(_S, 1)[:_SC_ROWS]

    gathered_sc = _sc_gather(cache, idx_sc)

    def _spec(k):
        return pl.BlockSpec(
            (1, 1, _CACHE_COLS),
            lambda i, p, k=k: (p[_SC_ROWS + _BLK * i + k], 0, 0),
        )

    grid_a = pltpu.PrefetchScalarGridSpec(
        num_scalar_prefetch=1,
        grid=(_A_STEPS,),
        in_specs=[_spec(k) for k in range(_BLK)],
        out_specs=pl.BlockSpec(
            (_BLK, _OUT_COLS), lambda i, p: (i + _SC_ROWS // _BLK, 0)
        ),
    )
    partial_out = pl.pallas_call(
        _a_body,
        grid_spec=grid_a,
        out_shape=jax.ShapeDtypeStruct((_S, _OUT_COLS), jnp.float32),
    )(parents, *([cache3] * _BLK))

    out = pl.pallas_call(
        _b_body,
        grid=(_B_STEPS,),
        in_specs=[
            pl.BlockSpec((_BLK, _CACHE_COLS), lambda i: (i, 0)),
            pl.BlockSpec(memory_space=pl.ANY),
        ],
        out_specs=pl.BlockSpec((_BLK, _OUT_COLS), lambda i: (i, 0)),
        out_shape=jax.ShapeDtypeStruct((_S, _OUT_COLS), jnp.float32),
        input_output_aliases={1: 0},
    )(gathered_sc, partial_out)
    return out.reshape(1, 1, _S, _OUT_COLS)


# speedup vs baseline: 1.3762x; 1.3762x over previous
import functools

import jax
import jax.numpy as jnp
from jax import lax
from jax.experimental import pallas as pl
from jax.experimental.pallas import tpu as pltpu
from jax.experimental.pallas import tpu_sc as plsc

_PREFIX = 32768
_S = 64
_CACHE_COLS = _PREFIX + _S * 16
_OUT_COLS = _PREFIX + _S
_NEG = jnp.finfo(jnp.float32).min
_NW = 32
_RPW = _S // _NW
_BLK = 32


@functools.partial(
    pl.kernel,
    out_type=jax.ShapeDtypeStruct((_S, _CACHE_COLS), jnp.bool_),
    mesh=plsc.VectorSubcoreMesh(core_axis_name="c", subcore_axis_name="s"),
    scratch_types=[
        pltpu.VMEM((_RPW,), jnp.int32),
        pltpu.VMEM((_RPW, _CACHE_COLS), jnp.bool_),
        pltpu.SemaphoreType.DMA,
    ],
)
def _sc_gather(table_hbm, idx_hbm, out_hbm, idx_v, rows_v, sem):
    wid = lax.axis_index("s") * 2 + lax.axis_index("c")
    base = wid * _RPW
    pltpu.sync_copy(idx_hbm.at[wid], idx_v)
    pltpu.async_copy(table_hbm.at[idx_v], rows_v, sem).wait()
    pltpu.sync_copy(rows_v, out_hbm.at[pl.ds(base, _RPW)])


def _convert_body(g_ref, out_ref):
    zero = jnp.float32(0.0)
    neg = jnp.float32(_NEG)
    row0 = _BLK * pl.program_id(0)
    ri = lax.broadcasted_iota(jnp.int32, (_BLK, _S), 0) + row0
    ci = lax.broadcasted_iota(jnp.int32, (_BLK, _S), 1)
    mask = jnp.concatenate([g_ref[:, :_PREFIX], ri == ci], axis=1)
    out_ref[...] = jnp.where(mask, zero, neg)


def kernel(parent_indices, tree_mask_cache, eye_block):
    del eye_block
    cache = tree_mask_cache.reshape(_S, _CACHE_COLS)
    idx = parent_indices.reshape(_NW, _RPW)

    gathered = _sc_gather(cache, idx)

    out = pl.pallas_call(
        _convert_body,
        grid=(_S // _BLK,),
        in_specs=[pl.BlockSpec((_BLK, _CACHE_COLS), lambda i: (i, 0))],
        out_specs=pl.BlockSpec((_BLK, _OUT_COLS), lambda i: (i, 0)),
        out_shape=jax.ShapeDtypeStruct((_S, _OUT_COLS), jnp.float32),
    )(gathered)
    return out.reshape(1, 1, _S, _OUT_COLS)
